# baseline (speedup 1.0000x reference)
import functools

import jax
import jax.numpy as jnp
from jax import lax
from jax.experimental import pallas as pl
from jax.experimental.pallas import tpu as pltpu
from jax.experimental.pallas import tpu_sc as plsc

OUT_GRID = 5
NVOX = OUT_GRID * OUT_GRID * OUT_GRID
L = 16
K = 128
NACC = 4


def _sc_pool(pts_hbm, feats_hbm, roip_hbm, out_hbm,
             pts_v, par_v, comb_v, didxa, didxb, fbufa, fbufb,
             acc0, acc1, acc2, acc3, ctr, sema, semb,
             *, n_pts, n_chan, rois_per_batch):
    b = lax.axis_index("c")
    s = lax.axis_index("s")
    qn = n_chan // L
    accs = [acc0, acc1, acc2, acc3]

    pltpu.sync_copy(pts_hbm.at[b], pts_v)

    @pl.when(s == 0)
    def _():
        ctr[0] = 0

    plsc.subcore_barrier()

    def do_roi(r):
        pltpu.sync_copy(roip_hbm.at[b, r], par_v)
        pv = par_v[...]
        cx, cy, cz = pv[0], pv[1], pv[2]
        hx, hy, hz = pv[3], pv[4], pv[5]
        ca, sa = pv[6], pv[7]
        bx, by, bz = pv[8], pv[9], pv[10]

        minus_inf = jnp.full((L,), -jnp.inf, jnp.float32)

        @plsc.parallel_loop(0, (128 * n_chan) // L, unroll=8)
        def _(i):
            acc0[pl.ds(i * L, L)] = minus_inf
            acc1[pl.ds(i * L, L)] = minus_inf
            acc2[pl.ds(i * L, L)] = minus_inf
            acc3[pl.ds(i * L, L)] = minus_inf

        @plsc.parallel_loop(0, n_pts // L, unroll=4, carry=jnp.int32(0))
        def cnt(i, cnt):
            px = pts_v[0, pl.ds(i * L, L)]
            py = pts_v[1, pl.ds(i * L, L)]
            pz = pts_v[2, pl.ds(i * L, L)]
            sx = px - cx
            sy = py - cy
            lz = pz - cz
            lx = sx * ca - sy * sa
            ly = sx * sa + sy * ca
            inb = ((jnp.abs(lx) < hx) & (jnp.abs(ly) < hy)
                   & (jnp.abs(lz) < hz))

            def vix(lv, hv, bv):
                t = ((lv + hv) / bv).astype(jnp.int32)
                return jnp.minimum(t, OUT_GRID - 1)

            vox = (vix(lx, hx, bx) * (OUT_GRID * OUT_GRID)
                   + vix(ly, hy, by) * OUT_GRID + vix(lz, hz, bz))
            pidx = lax.iota(jnp.int32, L) + (i * L + b * n_pts)
            packed = (pidx << 7) | vox
            xi = inb.astype(jnp.int32)
            c = jnp.cumsum(xi)
            pos = (cnt + c) - xi
            plsc.store_scatter(comb_v, [pos], packed, mask=inb)
            return cnt + plsc.all_reduce_population_count(inb)[0]

        dummy = jnp.full((L,), NVOX, jnp.int32)
        for t in range(K // L):
            comb_v[pl.ds(cnt + t * L, L)] = dummy

        nchunks = (cnt + (K - 1)) // K

        def stage_didx(base, didx):
            for t in range(K // L):
                pk = comb_v[pl.ds(base + t * L, L)]
                didx[pl.ds(t * L, L)] = pk >> 7

        @pl.when(nchunks > 0)
        def _():
            stage_didx(0, didxa)
            pltpu.async_copy(feats_hbm.at[didxa], fbufa, sema)

        def process(base, fb):
            for t in range(K // L):
                offs = (comb_v[pl.ds(base + t * L, L)] & (128 - 1)) * n_chan
                for k in range(L):
                    off = offs[k]
                    a_ref = accs[k % NACC]
                    for q in range(qn):
                        a = a_ref[pl.ds(off + q * L, L)]
                        f = fb[t * L + k, pl.ds(q * L, L)]
                        a_ref[pl.ds(off + q * L, L)] = jnp.maximum(a, f)

        def chunk_body(m, _):
            base = m * K

            @pl.when(m % 2 == 0)
            def _():
                @pl.when(m + 1 < nchunks)
                def _():
                    stage_didx(base + K, didxb)
                    pltpu.async_copy(feats_hbm.at[didxb], fbufb, semb)

                pltpu.make_async_copy(feats_hbm.at[didxa],
                                      fbufa, sema).wait()
                process(base, fbufa)

            @pl.when(m % 2 == 1)
            def _():
                @pl.when(m + 1 < nchunks)
                def _():
                    stage_didx(base + K, didxa)
                    pltpu.async_copy(feats_hbm.at[didxa], fbufa, sema)

                pltpu.make_async_copy(feats_hbm.at[didxb],
                                      fbufb, semb).wait()
                process(base, fbufb)

            return 0

        lax.fori_loop(0, nchunks, chunk_body, 0)

        @plsc.parallel_loop(0, (NVOX * n_chan) // L, unroll=8)
        def _(i):
            v01 = jnp.maximum(acc0[pl.ds(i * L, L)], acc1[pl.ds(i * L, L)])
            v23 = jnp.maximum(acc2[pl.ds(i * L, L)], acc3[pl.ds(i * L, L)])
            v = jnp.maximum(v01, v23)
            acc0[pl.ds(i * L, L)] = jnp.where(v == -jnp.inf, 0.0, v)

        rid = b * rois_per_batch + r
        pltpu.sync_copy(acc0.at[pl.ds(0, NVOX * n_chan)], out_hbm.at[rid])

    r0 = plsc.fetch_and_add(ctr.at[0], 1, subcore_id=0)

    def steal_cond(r):
        return r < rois_per_batch

    def steal_body(r):
        do_roi(r)
        return plsc.fetch_and_add(ctr.at[0], 1, subcore_id=0)

    lax.while_loop(steal_cond, steal_body, r0)


def kernel(points_xyz, features, rois):
    B, N, _ = points_xyz.shape
    C = features.shape[1]
    R = rois.shape[1]

    pts = jnp.transpose(points_xyz, (0, 2, 1))
    featsT = jnp.transpose(features, (0, 2, 1))
    featsT = featsT.reshape(B * N, C)

    center = rois[..., 0:3]
    dims = rois[..., 3:6]
    ry = rois[..., 6]
    half = dims / 2.0
    ca = jnp.cos(-ry)[..., None]
    sa = jnp.sin(-ry)[..., None]
    binsz = dims / OUT_GRID
    pad = jnp.zeros((B, R, 5), jnp.float32)
    roip = jnp.concatenate([center, half, ca, sa, binsz, pad], axis=-1)

    mesh = plsc.VectorSubcoreMesh(core_axis_name="c", subcore_axis_name="s",
                                  num_cores=2, num_subcores=16)
    body = functools.partial(_sc_pool, n_pts=N, n_chan=C, rois_per_batch=R)
    sc = pl.kernel(
        body,
        out_type=jax.ShapeDtypeStruct((B * R, NVOX * C), jnp.float32),
        mesh=mesh,
        compiler_params=pltpu.CompilerParams(needs_layout_passes=False,
                                             use_tc_tiling_on_sc=False),
        scratch_types=[
            pltpu.VMEM((3, N), jnp.float32),
            pltpu.VMEM((L,), jnp.float32),
            pltpu.VMEM((N + K,), jnp.int32),
            pltpu.VMEM((K,), jnp.int32),
            pltpu.VMEM((K,), jnp.int32),
            pltpu.VMEM((K, C), jnp.float32),
            pltpu.VMEM((K, C), jnp.float32),
            pltpu.VMEM((128 * C,), jnp.float32),
            pltpu.VMEM((128 * C,), jnp.float32),
            pltpu.VMEM((128 * C,), jnp.float32),
            pltpu.VMEM((128 * C,), jnp.float32),
            pltpu.SMEM((1,), jnp.int32),
            pltpu.SemaphoreType.DMA,
            pltpu.SemaphoreType.DMA,
        ],
    )
    out = sc(pts, featsT, roip)
    out = out.reshape(B * R, NVOX, C)
    return jnp.transpose(out, (0, 2, 1))


# speedup vs baseline: 1.2294x; 1.2294x over previous
import functools

import jax
import jax.numpy as jnp
from jax import lax
from jax.experimental import pallas as pl
from jax.experimental.pallas import tpu as pltpu
from jax.experimental.pallas import tpu_sc as plsc

OUT_GRID = 5
NVOX = OUT_GRID * OUT_GRID * OUT_GRID
L = 16
K = 128
NACC = 4


def _sc_pool(pts_hbm, feats_hbm, roip_hbm, out_hbm,
             pts_v, par_v, comb_v, didxa, didxb, fbufa, fbufb,
             acc0, acc1, acc2, acc3, ctr, sema, semb,
             *, n_pts, n_chan, rois_per_batch):
    b = lax.axis_index("c")
    s = lax.axis_index("s")
    qn = n_chan // L
    accs = [acc0, acc1, acc2, acc3]

    pltpu.sync_copy(pts_hbm.at[b], pts_v)

    @pl.when(s == 0)
    def _():
        ctr[0] = 0

    plsc.subcore_barrier()

    def do_roi(r):
        pltpu.sync_copy(roip_hbm.at[b, r], par_v)
        pv = par_v[...]
        cx, cy, cz = pv[0], pv[1], pv[2]
        hx, hy, hz = pv[3], pv[4], pv[5]
        ca, sa = pv[6], pv[7]
        bx, by, bz = pv[8], pv[9], pv[10]

        minus_inf = jnp.full((L,), -jnp.inf, jnp.float32)

        @plsc.parallel_loop(0, (128 * n_chan) // L, unroll=8)
        def _(i):
            acc0[pl.ds(i * L, L)] = minus_inf
            acc1[pl.ds(i * L, L)] = minus_inf
            acc2[pl.ds(i * L, L)] = minus_inf
            acc3[pl.ds(i * L, L)] = minus_inf

        @plsc.parallel_loop(0, n_pts // L, unroll=4, carry=jnp.int32(0))
        def cnt(i, cnt):
            px = pts_v[0, pl.ds(i * L, L)]
            py = pts_v[1, pl.ds(i * L, L)]
            pz = pts_v[2, pl.ds(i * L, L)]
            sx = px - cx
            sy = py - cy
            lz = pz - cz
            lx = sx * ca - sy * sa
            ly = sx * sa + sy * ca
            inb = ((jnp.abs(lx) < hx) & (jnp.abs(ly) < hy)
                   & (jnp.abs(lz) < hz))

            def vix(lv, hv, bv):
                t = ((lv + hv) / bv).astype(jnp.int32)
                return jnp.minimum(t, OUT_GRID - 1)

            vox = (vix(lx, hx, bx) * (OUT_GRID * OUT_GRID)
                   + vix(ly, hy, by) * OUT_GRID + vix(lz, hz, bz))
            pidx = lax.iota(jnp.int32, L) + (i * L + b * n_pts)
            packed = (pidx << 7) | vox
            xi = inb.astype(jnp.int32)
            c = jnp.cumsum(xi)
            pos = (cnt + c) - xi
            plsc.store_scatter(comb_v, [pos], packed, mask=inb)
            return cnt + plsc.all_reduce_population_count(inb)[0]

        dummy = jnp.full((L,), NVOX, jnp.int32)
        for t in range(K // L):
            comb_v[pl.ds(cnt + t * L, L)] = dummy

        nchunks = (cnt + (K - 1)) // K

        def stage_didx(base, didx):
            for t in range(K // L):
                pk = comb_v[pl.ds(base + t * L, L)]
                didx[pl.ds(t * L, L)] = pk >> 7

        @pl.when(nchunks > 0)
        def _():
            stage_didx(0, didxa)
            pltpu.async_copy(feats_hbm.at[didxa], fbufa, sema)

        def process(base, fb):
            for t in range(K // L):
                offs = (comb_v[pl.ds(base + t * L, L)] & (128 - 1)) * n_chan
                for k in range(0, L, 2):
                    offa = offs[k]
                    offb = offs[k + 1]
                    ra = accs[k % NACC]
                    rb = accs[(k + 1) % NACC]
                    av = [ra[pl.ds(offa + q * L, L)] for q in range(qn)]
                    bv = [rb[pl.ds(offb + q * L, L)] for q in range(qn)]
                    fa = [fb[t * L + k, pl.ds(q * L, L)] for q in range(qn)]
                    fbv = [fb[t * L + k + 1, pl.ds(q * L, L)]
                           for q in range(qn)]
                    for q in range(qn):
                        ra[pl.ds(offa + q * L, L)] = jnp.maximum(av[q], fa[q])
                    for q in range(qn):
                        rb[pl.ds(offb + q * L, L)] = jnp.maximum(bv[q],
                                                                 fbv[q])

        def chunk_body(m, _):
            base = m * K

            @pl.when(m % 2 == 0)
            def _():
                @pl.when(m + 1 < nchunks)
                def _():
                    stage_didx(base + K, didxb)
                    pltpu.async_copy(feats_hbm.at[didxb], fbufb, semb)

                pltpu.make_async_copy(feats_hbm.at[didxa],
                                      fbufa, sema).wait()
                process(base, fbufa)

            @pl.when(m % 2 == 1)
            def _():
                @pl.when(m + 1 < nchunks)
                def _():
                    stage_didx(base + K, didxa)
                    pltpu.async_copy(feats_hbm.at[didxa], fbufa, sema)

                pltpu.make_async_copy(feats_hbm.at[didxb],
                                      fbufb, semb).wait()
                process(base, fbufb)

            return 0

        lax.fori_loop(0, nchunks, chunk_body, 0)

        @plsc.parallel_loop(0, (NVOX * n_chan) // L, unroll=8)
        def _(i):
            v01 = jnp.maximum(acc0[pl.ds(i * L, L)], acc1[pl.ds(i * L, L)])
            v23 = jnp.maximum(acc2[pl.ds(i * L, L)], acc3[pl.ds(i * L, L)])
            v = jnp.maximum(v01, v23)
            acc0[pl.ds(i * L, L)] = jnp.where(v == -jnp.inf, 0.0, v)

        rid = b * rois_per_batch + r
        pltpu.sync_copy(acc0.at[pl.ds(0, NVOX * n_chan)], out_hbm.at[rid])

    r0 = plsc.fetch_and_add(ctr.at[0], 1, subcore_id=0)

    def steal_cond(r):
        return r < rois_per_batch

    def steal_body(r):
        do_roi(r)
        return plsc.fetch_and_add(ctr.at[0], 1, subcore_id=0)

    lax.while_loop(steal_cond, steal_body, r0)


def kernel(points_xyz, features, rois):
    B, N, _ = points_xyz.shape
    C = features.shape[1]
    R = rois.shape[1]

    pts = jnp.transpose(points_xyz, (0, 2, 1))
    featsT = jnp.transpose(features, (0, 2, 1))
    featsT = featsT.reshape(B * N, C)

    center = rois[..., 0:3]
    dims = rois[..., 3:6]
    ry = rois[..., 6]
    half = dims / 2.0
    ca = jnp.cos(-ry)[..., None]
    sa = jnp.sin(-ry)[..., None]
    binsz = dims / OUT_GRID
    pad = jnp.zeros((B, R, 5), jnp.float32)
    roip = jnp.concatenate([center, half, ca, sa, binsz, pad], axis=-1)

    mesh = plsc.VectorSubcoreMesh(core_axis_name="c", subcore_axis_name="s",
                                  num_cores=2, num_subcores=16)
    body = functools.partial(_sc_pool, n_pts=N, n_chan=C, rois_per_batch=R)
    sc = pl.kernel(
        body,
        out_type=jax.ShapeDtypeStruct((B * R, NVOX * C), jnp.float32),
        mesh=mesh,
        compiler_params=pltpu.CompilerParams(needs_layout_passes=False,
                                             use_tc_tiling_on_sc=False),
        scratch_types=[
            pltpu.VMEM((3, N), jnp.float32),
            pltpu.VMEM((L,), jnp.float32),
            pltpu.VMEM((N + K,), jnp.int32),
            pltpu.VMEM((K,), jnp.int32),
            pltpu.VMEM((K,), jnp.int32),
            pltpu.VMEM((K, C), jnp.float32),
            pltpu.VMEM((K, C), jnp.float32),
            pltpu.VMEM((128 * C,), jnp.float32),
            pltpu.VMEM((128 * C,), jnp.float32),
            pltpu.VMEM((128 * C,), jnp.float32),
            pltpu.VMEM((128 * C,), jnp.float32),
            pltpu.SMEM((1,), jnp.int32),
            pltpu.SemaphoreType.DMA,
            pltpu.SemaphoreType.DMA,
        ],
    )
    out = sc(pts, featsT, roip)
    out = out.reshape(B * R, NVOX, C)
    return jnp.transpose(out, (0, 2, 1))


# speedup vs baseline: 1.2771x; 1.0388x over previous
import functools

import jax
import jax.numpy as jnp
from jax import lax
from jax.experimental import pallas as pl
from jax.experimental.pallas import tpu as pltpu
from jax.experimental.pallas import tpu_sc as plsc

OUT_GRID = 5
NVOX = OUT_GRID * OUT_GRID * OUT_GRID
L = 16
K = 128
NACC = 4


def _sc_pool(pts_hbm, feats_hbm, roip_hbm, out_hbm,
             pts_v, par_v, comb_v, didxa, didxb, fbufa, fbufb,
             acc0, acc1, acc2, acc3, ctr, sema, semb,
             *, n_pts, n_chan, rois_per_batch):
    b = lax.axis_index("c")
    s = lax.axis_index("s")
    qn = n_chan // L
    accs = [acc0, acc1, acc2, acc3]

    pltpu.sync_copy(pts_hbm.at[b], pts_v)

    @pl.when(s == 0)
    def _():
        ctr[0] = 0

    plsc.subcore_barrier()

    def do_roi(r):
        pltpu.sync_copy(roip_hbm.at[b, r], par_v)
        pv = par_v[...]
        cx, cy, cz = pv[0], pv[1], pv[2]
        hx, hy, hz = pv[3], pv[4], pv[5]
        ca, sa = pv[6], pv[7]
        bx, by, bz = pv[8], pv[9], pv[10]

        minus_inf = jnp.full((L,), -jnp.inf, jnp.float32)

        @plsc.parallel_loop(0, (128 * n_chan) // L, unroll=8)
        def _(i):
            acc0[pl.ds(i * L, L)] = minus_inf
            acc1[pl.ds(i * L, L)] = minus_inf
            acc2[pl.ds(i * L, L)] = minus_inf
            acc3[pl.ds(i * L, L)] = minus_inf

        @plsc.parallel_loop(0, n_pts // L, unroll=4, carry=jnp.int32(0))
        def cnt(i, cnt):
            px = pts_v[0, pl.ds(i * L, L)]
            py = pts_v[1, pl.ds(i * L, L)]
            pz = pts_v[2, pl.ds(i * L, L)]
            sx = px - cx
            sy = py - cy
            lz = pz - cz
            lx = sx * ca - sy * sa
            ly = sx * sa + sy * ca
            inb = ((jnp.abs(lx) < hx) & (jnp.abs(ly) < hy)
                   & (jnp.abs(lz) < hz))

            def vix(lv, hv, bv):
                t = ((lv + hv) / bv).astype(jnp.int32)
                return jnp.minimum(t, OUT_GRID - 1)

            vox = (vix(lx, hx, bx) * (OUT_GRID * OUT_GRID)
                   + vix(ly, hy, by) * OUT_GRID + vix(lz, hz, bz))
            pidx = lax.iota(jnp.int32, L) + (i * L + b * n_pts)
            packed = (pidx << 7) | vox
            xi = inb.astype(jnp.int32)
            c = jnp.cumsum(xi)
            pos = (cnt + c) - xi
            plsc.store_scatter(comb_v, [pos], packed, mask=inb)
            return cnt + plsc.all_reduce_population_count(inb)[0]

        dummy = jnp.full((L,), NVOX, jnp.int32)
        for t in range(K // L):
            comb_v[pl.ds(cnt + t * L, L)] = dummy

        nchunks = (cnt + (K - 1)) // K

        def stage_didx(base, didx):
            for t in range(K // L):
                pk = comb_v[pl.ds(base + t * L, L)]
                didx[pl.ds(t * L, L)] = pk >> 7

        @pl.when(nchunks > 0)
        def _():
            stage_didx(0, didxa)
            pltpu.async_copy(feats_hbm.at[didxa], fbufa, sema)

        def process(base, fb):
            for t in range(K // L):
                offs = (comb_v[pl.ds(base + t * L, L)] & (128 - 1)) * n_chan
                for k in range(0, L, 2):
                    offa = offs[k]
                    offb = offs[k + 1]
                    ra = accs[k % NACC]
                    rb = accs[(k + 1) % NACC]
                    av = [ra[pl.ds(offa + q * L, L)] for q in range(qn)]
                    bv = [rb[pl.ds(offb + q * L, L)] for q in range(qn)]
                    fa = [fb[t * L + k, pl.ds(q * L, L)] for q in range(qn)]
                    fbv = [fb[t * L + k + 1, pl.ds(q * L, L)]
                           for q in range(qn)]
                    for q in range(qn):
                        ra[pl.ds(offa + q * L, L)] = jnp.maximum(av[q], fa[q])
                    for q in range(qn):
                        rb[pl.ds(offb + q * L, L)] = jnp.maximum(bv[q],
                                                                 fbv[q])

        def chunk_body(m, _):
            base = m * K

            @pl.when(m % 2 == 0)
            def _():
                @pl.when(m + 1 < nchunks)
                def _():
                    stage_didx(base + K, didxb)
                    pltpu.async_copy(feats_hbm.at[didxb], fbufb, semb)

                pltpu.make_async_copy(feats_hbm.at[didxa],
                                      fbufa, sema).wait()
                process(base, fbufa)

            @pl.when(m % 2 == 1)
            def _():
                @pl.when(m + 1 < nchunks)
                def _():
                    stage_didx(base + K, didxa)
                    pltpu.async_copy(feats_hbm.at[didxa], fbufa, sema)

                pltpu.make_async_copy(feats_hbm.at[didxb],
                                      fbufb, semb).wait()
                process(base, fbufb)

            return 0

        lax.fori_loop(0, nchunks, chunk_body, 0)

        @plsc.parallel_loop(0, (NVOX * n_chan) // L, unroll=8)
        def _(i):
            v01 = jnp.maximum(acc0[pl.ds(i * L, L)], acc1[pl.ds(i * L, L)])
            v23 = jnp.maximum(acc2[pl.ds(i * L, L)], acc3[pl.ds(i * L, L)])
            v = jnp.maximum(v01, v23)
            acc0[pl.ds(i * L, L)] = jnp.where(v == -jnp.inf, 0.0, v)

        rid = b * rois_per_batch + pv[11].astype(jnp.int32)
        pltpu.sync_copy(acc0.at[pl.ds(0, NVOX * n_chan)], out_hbm.at[rid])

    r0 = plsc.fetch_and_add(ctr.at[0], 1, subcore_id=0)

    def steal_cond(r):
        return r < rois_per_batch

    def steal_body(r):
        do_roi(r)
        return plsc.fetch_and_add(ctr.at[0], 1, subcore_id=0)

    lax.while_loop(steal_cond, steal_body, r0)


def kernel(points_xyz, features, rois):
    B, N, _ = points_xyz.shape
    C = features.shape[1]
    R = rois.shape[1]

    pts = jnp.transpose(points_xyz, (0, 2, 1))
    featsT = jnp.transpose(features, (0, 2, 1))
    featsT = featsT.reshape(B * N, C)

    center = rois[..., 0:3]
    dims = rois[..., 3:6]
    ry = rois[..., 6]
    half = dims / 2.0
    ca = jnp.cos(-ry)[..., None]
    sa = jnp.sin(-ry)[..., None]
    binsz = dims / OUT_GRID
    vol = dims[..., 0] * dims[..., 1] * dims[..., 2]
    order = jnp.argsort(-vol, axis=1)
    ridf = jnp.broadcast_to(
        jnp.arange(R, dtype=jnp.float32)[None, :, None], (B, R, 1))
    pad = jnp.zeros((B, R, 4), jnp.float32)
    roip = jnp.concatenate([center, half, ca, sa, binsz, ridf, pad], axis=-1)
    roip = jnp.take_along_axis(roip, order[..., None], axis=1)

    mesh = plsc.VectorSubcoreMesh(core_axis_name="c", subcore_axis_name="s",
                                  num_cores=2, num_subcores=16)
    body = functools.partial(_sc_pool, n_pts=N, n_chan=C, rois_per_batch=R)
    sc = pl.kernel(
        body,
        out_type=jax.ShapeDtypeStruct((B * R, NVOX * C), jnp.float32),
        mesh=mesh,
        compiler_params=pltpu.CompilerParams(needs_layout_passes=False,
                                             use_tc_tiling_on_sc=False),
        scratch_types=[
            pltpu.VMEM((3, N), jnp.float32),
            pltpu.VMEM((L,), jnp.float32),
            pltpu.VMEM((N + K,), jnp.int32),
            pltpu.VMEM((K,), jnp.int32),
            pltpu.VMEM((K,), jnp.int32),
            pltpu.VMEM((K, C), jnp.float32),
            pltpu.VMEM((K, C), jnp.float32),
            pltpu.VMEM((128 * C,), jnp.float32),
            pltpu.VMEM((128 * C,), jnp.float32),
            pltpu.VMEM((128 * C,), jnp.float32),
            pltpu.VMEM((128 * C,), jnp.float32),
            pltpu.SMEM((1,), jnp.int32),
            pltpu.SemaphoreType.DMA,
            pltpu.SemaphoreType.DMA,
        ],
    )
    out = sc(pts, featsT, roip)
    out = out.reshape(B * R, NVOX, C)
    return jnp.transpose(out, (0, 2, 1))


# speedup vs baseline: 1.4264x; 1.1169x over previous
import functools

import jax
import jax.numpy as jnp
from jax import lax
from jax.experimental import pallas as pl
from jax.experimental.pallas import tpu as pltpu
from jax.experimental.pallas import tpu_sc as plsc

OUT_GRID = 5
NVOX = OUT_GRID * OUT_GRID * OUT_GRID
L = 16
K = 128
NACC = 4


def _sc_pool(pts_hbm, feats_hbm, roip_hbm, out_hbm,
             pts_v, par_v, comb_v, didxa, didxb, didxc, fbufa, fbufb, fbufc,
             acc0, acc1, acc2, acc3, ctr, sema, semb, semc,
             *, n_pts, n_chan, rois_per_batch):
    b = lax.axis_index("c")
    s = lax.axis_index("s")
    qn = n_chan // L
    accs = [acc0, acc1, acc2, acc3]

    pltpu.sync_copy(pts_hbm.at[b], pts_v)

    @pl.when(s == 0)
    def _():
        ctr[0] = 0

    plsc.subcore_barrier()

    def do_roi(r):
        pltpu.sync_copy(roip_hbm.at[b, r], par_v)
        pv = par_v[...]
        cx, cy, cz = pv[0], pv[1], pv[2]
        hx, hy, hz = pv[3], pv[4], pv[5]
        ca, sa = pv[6], pv[7]
        bx, by, bz = pv[8], pv[9], pv[10]

        minus_inf = jnp.full((L,), -jnp.inf, jnp.float32)

        @plsc.parallel_loop(0, (128 * n_chan) // L, unroll=8)
        def _(i):
            acc0[pl.ds(i * L, L)] = minus_inf
            acc1[pl.ds(i * L, L)] = minus_inf
            acc2[pl.ds(i * L, L)] = minus_inf
            acc3[pl.ds(i * L, L)] = minus_inf

        @plsc.parallel_loop(0, n_pts // L, unroll=4, carry=jnp.int32(0))
        def cnt(i, cnt):
            px = pts_v[0, pl.ds(i * L, L)]
            py = pts_v[1, pl.ds(i * L, L)]
            pz = pts_v[2, pl.ds(i * L, L)]
            sx = px - cx
            sy = py - cy
            lz = pz - cz
            lx = sx * ca - sy * sa
            ly = sx * sa + sy * ca
            inb = ((jnp.abs(lx) < hx) & (jnp.abs(ly) < hy)
                   & (jnp.abs(lz) < hz))

            def vix(lv, hv, bv):
                t = ((lv + hv) / bv).astype(jnp.int32)
                return jnp.minimum(t, OUT_GRID - 1)

            vox = (vix(lx, hx, bx) * (OUT_GRID * OUT_GRID)
                   + vix(ly, hy, by) * OUT_GRID + vix(lz, hz, bz))
            pidx = lax.iota(jnp.int32, L) + (i * L + b * n_pts)
            packed = (pidx << 7) | vox
            xi = inb.astype(jnp.int32)
            c = jnp.cumsum(xi)
            pos = (cnt + c) - xi
            plsc.store_scatter(comb_v, [pos], packed, mask=inb)
            return cnt + plsc.all_reduce_population_count(inb)[0]

        dummy = jnp.full((L,), NVOX, jnp.int32)
        for t in range(K // L):
            comb_v[pl.ds(cnt + t * L, L)] = dummy

        nchunks = (cnt + (K - 1)) // K

        def stage_didx(base, didx):
            for t in range(K // L):
                pk = comb_v[pl.ds(base + t * L, L)]
                didx[pl.ds(t * L, L)] = pk >> 7

        bufs = [(didxa, fbufa, sema), (didxb, fbufb, semb),
                (didxc, fbufc, semc)]

        @pl.when(nchunks > 0)
        def _():
            stage_didx(0, didxa)
            pltpu.async_copy(feats_hbm.at[didxa], fbufa, sema)

        @pl.when(nchunks > 1)
        def _():
            stage_didx(K, didxb)
            pltpu.async_copy(feats_hbm.at[didxb], fbufb, semb)

        def process(base, fb):
            for t in range(K // L):
                offs = (comb_v[pl.ds(base + t * L, L)] & (128 - 1)) * n_chan
                for k in range(0, L, 2):
                    offa = offs[k]
                    offb = offs[k + 1]
                    ra = accs[k % NACC]
                    rb = accs[(k + 1) % NACC]
                    av = [ra[pl.ds(offa + q * L, L)] for q in range(qn)]
                    bv = [rb[pl.ds(offb + q * L, L)] for q in range(qn)]
                    fa = [fb[t * L + k, pl.ds(q * L, L)] for q in range(qn)]
                    fbv = [fb[t * L + k + 1, pl.ds(q * L, L)]
                           for q in range(qn)]
                    for q in range(qn):
                        ra[pl.ds(offa + q * L, L)] = jnp.maximum(av[q], fa[q])
                    for q in range(qn):
                        rb[pl.ds(offb + q * L, L)] = jnp.maximum(bv[q],
                                                                 fbv[q])

        def chunk_body(m, _):
            base = m * K
            for j in range(3):
                @pl.when(m % 3 == j)
                def _(j=j):
                    didx_n, fbuf_n, sem_n = bufs[(j + 2) % 3]

                    @pl.when(m + 2 < nchunks)
                    def _():
                        stage_didx(base + 2 * K, didx_n)
                        pltpu.async_copy(feats_hbm.at[didx_n], fbuf_n, sem_n)

                    didx_c, fbuf_c, sem_c = bufs[j]
                    pltpu.make_async_copy(feats_hbm.at[didx_c],
                                          fbuf_c, sem_c).wait()
                    process(base, fbuf_c)

            return 0

        lax.fori_loop(0, nchunks, chunk_body, 0)

        @plsc.parallel_loop(0, (NVOX * n_chan) // L, unroll=8)
        def _(i):
            v01 = jnp.maximum(acc0[pl.ds(i * L, L)], acc1[pl.ds(i * L, L)])
            v23 = jnp.maximum(acc2[pl.ds(i * L, L)], acc3[pl.ds(i * L, L)])
            v = jnp.maximum(v01, v23)
            acc0[pl.ds(i * L, L)] = jnp.where(v == -jnp.inf, 0.0, v)

        rid = b * rois_per_batch + pv[11].astype(jnp.int32)
        pltpu.sync_copy(acc0.at[pl.ds(0, NVOX * n_chan)], out_hbm.at[rid])

    r0 = plsc.fetch_and_add(ctr.at[0], 1, subcore_id=0)

    def steal_cond(r):
        return r < rois_per_batch

    def steal_body(r):
        do_roi(r)
        return plsc.fetch_and_add(ctr.at[0], 1, subcore_id=0)

    lax.while_loop(steal_cond, steal_body, r0)


def kernel(points_xyz, features, rois):
    B, N, _ = points_xyz.shape
    C = features.shape[1]
    R = rois.shape[1]

    pts = jnp.transpose(points_xyz, (0, 2, 1))
    featsT = jnp.transpose(features, (0, 2, 1))
    featsT = featsT.reshape(B * N, C)

    center = rois[..., 0:3]
    dims = rois[..., 3:6]
    ry = rois[..., 6]
    half = dims / 2.0
    ca = jnp.cos(-ry)[..., None]
    sa = jnp.sin(-ry)[..., None]
    binsz = dims / OUT_GRID
    vol = dims[..., 0] * dims[..., 1] * dims[..., 2]
    order = jnp.argsort(-vol, axis=1)
    ridf = jnp.broadcast_to(
        jnp.arange(R, dtype=jnp.float32)[None, :, None], (B, R, 1))
    pad = jnp.zeros((B, R, 4), jnp.float32)
    roip = jnp.concatenate([center, half, ca, sa, binsz, ridf, pad], axis=-1)
    roip = jnp.take_along_axis(roip, order[..., None], axis=1)

    mesh = plsc.VectorSubcoreMesh(core_axis_name="c", subcore_axis_name="s",
                                  num_cores=2, num_subcores=16)
    body = functools.partial(_sc_pool, n_pts=N, n_chan=C, rois_per_batch=R)
    sc = pl.kernel(
        body,
        out_type=jax.ShapeDtypeStruct((B * R, NVOX * C), jnp.float32),
        mesh=mesh,
        compiler_params=pltpu.CompilerParams(needs_layout_passes=False,
                                             use_tc_tiling_on_sc=False),
        scratch_types=[
            pltpu.VMEM((3, N), jnp.float32),
            pltpu.VMEM((L,), jnp.float32),
            pltpu.VMEM((N + K,), jnp.int32),
            pltpu.VMEM((K,), jnp.int32),
            pltpu.VMEM((K,), jnp.int32),
            pltpu.VMEM((K,), jnp.int32),
            pltpu.VMEM((K, C), jnp.float32),
            pltpu.VMEM((K, C), jnp.float32),
            pltpu.VMEM((K, C), jnp.float32),
            pltpu.VMEM((128 * C,), jnp.float32),
            pltpu.VMEM((128 * C,), jnp.float32),
            pltpu.VMEM((128 * C,), jnp.float32),
            pltpu.VMEM((128 * C,), jnp.float32),
            pltpu.SMEM((1,), jnp.int32),
            pltpu.SemaphoreType.DMA,
            pltpu.SemaphoreType.DMA,
            pltpu.SemaphoreType.DMA,
        ],
    )
    out = sc(pts, featsT, roip)
    out = out.reshape(B * R, NVOX, C)
    return jnp.transpose(out, (0, 2, 1))
